# Initial kernel scaffold; baseline (speedup 1.0000x reference)
import jax
import jax.numpy as jnp
from jax.experimental import pallas as pl


def kernel(data, time_stamps, pe):
    raise NotImplementedError("write your pallas kernel here")


# speedup vs baseline: 2.5415x; 2.5415x over previous
import functools

import jax
import jax.numpy as jnp
from jax import lax
from jax.experimental import pallas as pl
from jax.experimental.pallas import tpu as pltpu
from jax.experimental.pallas import tpu_sc as plsc

NC = 2
NS = 16
NW = NC * NS
L = 16
CHUNK = 512
GCHUNK = 128


def _tec_body(data_hbm, ts_hbm, pe_hbm, out_hbm, ts_v, idx_v, pe_v, data_v,
              sem_d, sem_g):
    rows = data_hbm.shape[0]
    rows_per_w = rows // NW
    wid = lax.axis_index("s") * NC + lax.axis_index("c")
    wbase = wid * rows_per_w
    num_chunks = rows_per_w // CHUNK
    n_g = CHUNK // GCHUNK

    def chunk_body(ci, carry):
        base = wbase + ci * CHUNK
        pltpu.sync_copy(ts_hbm.at[pl.ds(base, CHUNK)], ts_v)
        dcp = pltpu.async_copy(data_hbm.at[pl.ds(base, CHUNK)], data_v, sem_d)

        for g in range(n_g):
            def div_body(i, _):
                t = ts_v[pl.ds(g * GCHUNK + i * L, L)].astype(jnp.float32)
                q = ((t + 0.5) * 0.01).astype(jnp.int32)
                idx_v[g, pl.ds(i * L, L)] = q
                return 0
            lax.fori_loop(0, GCHUNK // L, div_body, 0)

        gcps = []
        for g in range(n_g):
            gcps.append(pltpu.async_copy(
                pe_hbm.at[idx_v.at[g]],
                pe_v.at[pl.ds(g * GCHUNK, GCHUNK)],
                sem_g))
        dcp.wait()
        for cp in gcps:
            cp.wait()

        def add_body(i, _):
            for j in range(4):
                s = pl.ds(j * L, L)
                data_v[i, s] = data_v[i, s] + pe_v[i, s]
            return 0
        lax.fori_loop(0, CHUNK, add_body, 0)

        pltpu.sync_copy(data_v, out_hbm.at[pl.ds(base, CHUNK)])
        return carry

    lax.fori_loop(0, num_chunks, chunk_body, 0)


@jax.jit
def kernel(data, time_stamps, pe):
    b, h, d = data.shape
    rows = b * h
    data2 = data.reshape(rows, d)
    ts = time_stamps.reshape(rows)

    mesh = plsc.VectorSubcoreMesh(
        core_axis_name="c", subcore_axis_name="s", num_cores=NC,
        num_subcores=NS)
    out = pl.kernel(
        _tec_body,
        out_type=jax.ShapeDtypeStruct((rows, d), jnp.float32),
        mesh=mesh,
        scratch_types=[
            pltpu.VMEM((CHUNK,), jnp.int32),
            pltpu.VMEM((CHUNK // GCHUNK, GCHUNK), jnp.int32),
            pltpu.VMEM((CHUNK, 64), jnp.float32),
            pltpu.VMEM((CHUNK, 64), jnp.float32),
            pltpu.SemaphoreType.DMA,
            pltpu.SemaphoreType.DMA,
        ],
        compiler_params=pltpu.CompilerParams(use_tc_tiling_on_sc=False),
    )(data2, ts, pe)
    return out.reshape(b, h, d)


# speedup vs baseline: 2.6707x; 1.0508x over previous
import functools

import jax
import jax.numpy as jnp
from jax import lax
from jax.experimental import pallas as pl
from jax.experimental.pallas import tpu as pltpu
from jax.experimental.pallas import tpu_sc as plsc

NC = 2
NS = 16
NW = NC * NS
L = 16
CHUNK = 512
GCHUNK = 128


def _tec_body(data_hbm, ts_hbm, pe_hbm, out_hbm, ts_v, idx_v, data_v,
              sem_d, sem_g):
    rows = data_hbm.shape[0]
    rows_per_w = rows // NW
    wid = lax.axis_index("s") * NC + lax.axis_index("c")
    wbase = wid * rows_per_w
    num_chunks = rows_per_w // CHUNK
    n_g = CHUNK // GCHUNK

    def chunk_body(ci, carry):
        base = wbase + ci * CHUNK
        pltpu.sync_copy(ts_hbm.at[pl.ds(base, CHUNK)], ts_v)
        dcp = pltpu.async_copy(data_hbm.at[pl.ds(base, CHUNK)], data_v, sem_d)

        for g in range(n_g):
            def div_body(i, _):
                t = ts_v[pl.ds(g * GCHUNK + i * L, L)].astype(jnp.float32)
                q = ((t + 0.5) * 0.01).astype(jnp.int32)
                idx_v[g, pl.ds(i * L, L)] = q
                return 0
            lax.fori_loop(0, GCHUNK // L, div_body, 0)

        dcp.wait()
        gcps = []
        for g in range(n_g):
            gcps.append(pltpu.async_copy(
                pe_hbm.at[idx_v.at[g]],
                data_v.at[pl.ds(g * GCHUNK, GCHUNK)],
                sem_g, add=True))
        for cp in gcps:
            cp.wait()

        pltpu.sync_copy(data_v, out_hbm.at[pl.ds(base, CHUNK)])
        return carry

    lax.fori_loop(0, num_chunks, chunk_body, 0)


@jax.jit
def kernel(data, time_stamps, pe):
    b, h, d = data.shape
    rows = b * h
    data2 = data.reshape(rows, d)
    ts = time_stamps.reshape(rows)

    mesh = plsc.VectorSubcoreMesh(
        core_axis_name="c", subcore_axis_name="s", num_cores=NC,
        num_subcores=NS)
    out = pl.kernel(
        _tec_body,
        out_type=jax.ShapeDtypeStruct((rows, d), jnp.float32),
        mesh=mesh,
        scratch_types=[
            pltpu.VMEM((CHUNK,), jnp.int32),
            pltpu.VMEM((CHUNK // GCHUNK, GCHUNK), jnp.int32),
            pltpu.VMEM((CHUNK, 64), jnp.float32),
            pltpu.SemaphoreType.DMA,
            pltpu.SemaphoreType.DMA,
        ],
        compiler_params=pltpu.CompilerParams(use_tc_tiling_on_sc=False),
    )(data2, ts, pe)
    return out.reshape(b, h, d)


# speedup vs baseline: 2.8462x; 1.0657x over previous
import functools

import jax
import jax.numpy as jnp
from jax import lax
from jax.experimental import pallas as pl
from jax.experimental.pallas import tpu as pltpu
from jax.experimental.pallas import tpu_sc as plsc

NC = 2
NS = 16
NW = NC * NS
L = 16
CHUNK = 512
GCHUNK = 128


def _tec_body(data_hbm, ts_hbm, pe_hbm, out_hbm, ts_all, idx_all, data_v,
              sem_in0, sem_in1, sem_g0, sem_g1, sem_out0, sem_out1):
    rows = data_hbm.shape[0]
    rows_per_w = rows // NW
    wid = lax.axis_index("s") * NC + lax.axis_index("c")
    wbase = wid * rows_per_w
    num_chunks = rows_per_w // CHUNK
    n_g = CHUNK // GCHUNK
    sem_in = (sem_in0, sem_in1)
    sem_g = (sem_g0, sem_g1)
    sem_out = (sem_out0, sem_out1)

    def in_desc(ci, b):
        return pltpu.make_async_copy(
            data_hbm.at[pl.ds(wbase + ci * CHUNK, CHUNK)], data_v.at[b],
            sem_in[b])

    def fire_in(ci, b):
        in_desc(ci, b).start()

    def fire_gathers(ci, b):
        in_desc(ci, b).wait()
        for g in range(n_g):
            pltpu.async_copy(
                pe_hbm.at[idx_all.at[ci * n_g + g]],
                data_v.at[b, pl.ds(g * GCHUNK, GCHUNK)],
                sem_g[b], add=True)

    def out_desc(ci, b):
        return pltpu.make_async_copy(
            data_v.at[b], out_hbm.at[pl.ds(wbase + ci * CHUNK, CHUNK)],
            sem_out[b])

    def fire_out(ci, b):
        for g in range(n_g):
            pltpu.make_async_copy(
                pe_hbm.at[idx_all.at[ci * n_g + g]],
                data_v.at[b, pl.ds(g * GCHUNK, GCHUNK)],
                sem_g[b]).wait()
        out_desc(ci, b).start()

    pltpu.sync_copy(ts_hbm.at[pl.ds(wbase, rows_per_w)], ts_all)
    fire_in(0, 0)
    fire_in(1, 1)

    def div_body(r, _):
        for c in range(GCHUNK // L):
            t = ts_all[pl.ds(r * GCHUNK + c * L, L)].astype(jnp.float32)
            idx_all[r, pl.ds(c * L, L)] = ((t + 0.5) * 0.01).astype(jnp.int32)
        return 0
    lax.fori_loop(0, rows_per_w // GCHUNK, div_body, 0)

    fire_gathers(0, 0)

    def pair_body(p, _):
        c0 = p * 2
        fire_out(c0, 0)
        fire_gathers(c0 + 1, 1)
        out_desc(c0, 0).wait()
        fire_in(c0 + 2, 0)
        fire_out(c0 + 1, 1)
        fire_gathers(c0 + 2, 0)
        out_desc(c0 + 1, 1).wait()
        fire_in(c0 + 3, 1)
        return 0

    lax.fori_loop(0, num_chunks // 2 - 1, pair_body, 0)

    cl = num_chunks - 2
    fire_out(cl, 0)
    fire_gathers(cl + 1, 1)
    out_desc(cl, 0).wait()
    fire_out(cl + 1, 1)
    out_desc(cl + 1, 1).wait()


@jax.jit
def kernel(data, time_stamps, pe):
    b, h, d = data.shape
    rows = b * h
    rows_per_w = rows // NW
    data2 = data.reshape(rows, d)
    ts = time_stamps.reshape(rows)

    mesh = plsc.VectorSubcoreMesh(
        core_axis_name="c", subcore_axis_name="s", num_cores=NC,
        num_subcores=NS)
    out = pl.kernel(
        _tec_body,
        out_type=jax.ShapeDtypeStruct((rows, d), jnp.float32),
        mesh=mesh,
        scratch_types=[
            pltpu.VMEM((rows_per_w,), jnp.int32),
            pltpu.VMEM((rows_per_w // GCHUNK, GCHUNK), jnp.int32),
            pltpu.VMEM((2, CHUNK, 64), jnp.float32),
            pltpu.SemaphoreType.DMA,
            pltpu.SemaphoreType.DMA,
            pltpu.SemaphoreType.DMA,
            pltpu.SemaphoreType.DMA,
            pltpu.SemaphoreType.DMA,
            pltpu.SemaphoreType.DMA,
        ],
        compiler_params=pltpu.CompilerParams(use_tc_tiling_on_sc=False),
    )(data2, ts, pe)
    return out.reshape(b, h, d)


# speedup vs baseline: 2.8472x; 1.0004x over previous
import functools

import jax
import jax.numpy as jnp
from jax import lax
from jax.experimental import pallas as pl
from jax.experimental.pallas import tpu as pltpu
from jax.experimental.pallas import tpu_sc as plsc

NC = 2
NS = 16
NW = NC * NS
L = 16
CB = 2


def _tec_body(data_hbm, ts_hbm, pe_hbm, out_hbm, ts_all, idx_all, data_v,
              sem_in0, sem_in1, sem_g0, sem_g1, sem_out0, sem_out1):
    batch, hist, d = data_hbm.shape
    b_per_w = batch // NW
    wid = lax.axis_index("s") * NC + lax.axis_index("c")
    wbase = wid * b_per_w
    num_steps = b_per_w // CB
    sem_in = (sem_in0, sem_in1)
    sem_g = (sem_g0, sem_g1)
    sem_out = (sem_out0, sem_out1)
    gsplit = []
    off = 0
    while off < hist:
        n = min(128, hist - off)
        gsplit.append((off, n))
        off += n

    def in_desc(si, b):
        return pltpu.make_async_copy(
            data_hbm.at[pl.ds(wbase + si * CB, CB)], data_v.at[b], sem_in[b])

    def fire_in(si, b):
        in_desc(si, b).start()

    def gather_args(si, b):
        args = []
        for bl in range(CB):
            for off, n in gsplit:
                args.append((
                    pe_hbm.at[idx_all.at[si * CB + bl, pl.ds(off, n)]],
                    data_v.at[b, bl, pl.ds(off, n)],
                ))
        return args

    def fire_gathers(si, b):
        in_desc(si, b).wait()
        for src, dst in gather_args(si, b):
            pltpu.async_copy(src, dst, sem_g[b], add=True)

    def out_desc(si, b):
        return pltpu.make_async_copy(
            data_v.at[b], out_hbm.at[pl.ds(wbase + si * CB, CB)], sem_out[b])

    def fire_out(si, b):
        for src, dst in gather_args(si, b):
            pltpu.make_async_copy(src, dst, sem_g[b]).wait()
        out_desc(si, b).start()

    pltpu.sync_copy(ts_hbm.at[pl.ds(wbase, b_per_w)], ts_all)
    fire_in(0, 0)
    fire_in(1, 1)

    nfull = hist // L
    tail = hist - nfull * L
    cols = [c * L for c in range(nfull)] + ([hist - L] if tail else [])

    def div_body(r, _):
        for c in cols:
            t = ts_all[r, pl.ds(c, L)].astype(jnp.float32)
            idx_all[r, pl.ds(c, L)] = ((t + 0.5) * 0.01).astype(jnp.int32)
        return 0
    lax.fori_loop(0, b_per_w, div_body, 0)

    fire_gathers(0, 0)

    def pair_body(p, _):
        s0 = p * 2
        fire_out(s0, 0)
        fire_gathers(s0 + 1, 1)
        out_desc(s0, 0).wait()
        fire_in(s0 + 2, 0)
        fire_out(s0 + 1, 1)
        fire_gathers(s0 + 2, 0)
        out_desc(s0 + 1, 1).wait()
        fire_in(s0 + 3, 1)
        return 0

    lax.fori_loop(0, num_steps // 2 - 1, pair_body, 0)

    sl = num_steps - 2
    fire_out(sl, 0)
    fire_gathers(sl + 1, 1)
    out_desc(sl, 0).wait()
    fire_out(sl + 1, 1)
    out_desc(sl + 1, 1).wait()


@jax.jit
def kernel(data, time_stamps, pe):
    b, h, d = data.shape
    b_per_w = b // NW

    mesh = plsc.VectorSubcoreMesh(
        core_axis_name="c", subcore_axis_name="s", num_cores=NC,
        num_subcores=NS)
    out = pl.kernel(
        _tec_body,
        out_type=jax.ShapeDtypeStruct((b, h, d), jnp.float32),
        mesh=mesh,
        scratch_types=[
            pltpu.VMEM((b_per_w, h), jnp.int32),
            pltpu.VMEM((b_per_w, h), jnp.int32),
            pltpu.VMEM((2, CB, h, d), jnp.float32),
            pltpu.SemaphoreType.DMA,
            pltpu.SemaphoreType.DMA,
            pltpu.SemaphoreType.DMA,
            pltpu.SemaphoreType.DMA,
            pltpu.SemaphoreType.DMA,
            pltpu.SemaphoreType.DMA,
        ],
        compiler_params=pltpu.CompilerParams(use_tc_tiling_on_sc=False),
    )(data, time_stamps, pe)
    return out


# speedup vs baseline: 3.3036x; 1.1603x over previous
import functools

import jax
import jax.numpy as jnp
from jax import lax
from jax.experimental import pallas as pl
from jax.experimental.pallas import tpu as pltpu
from jax.experimental.pallas import tpu_sc as plsc

NC = 2
NS = 16
NW = NC * NS
L = 16


def _tec_body(data_hbm, ts_hbm, pe_hbm, out_hbm, ts_s, idx_s, data_in,
              pe_rows, sem_in0, sem_in1, sem_ts0, sem_ts1, sem_g0, sem_g1,
              sem_out0, sem_out1):
    batch, hist, d = data_hbm.shape
    b_per_w = batch // NW
    wid = lax.axis_index("s") * NC + lax.axis_index("c")
    wbase = wid * b_per_w
    sem_in = (sem_in0, sem_in1)
    sem_ts = (sem_ts0, sem_ts1)
    sem_g = (sem_g0, sem_g1)
    sem_out = (sem_out0, sem_out1)
    gsplit = []
    off = 0
    while off < hist:
        n = min(128, hist - off)
        gsplit.append((off, n))
        off += n
    nfull = hist // L
    cols = [c * L for c in range(nfull)] + ([hist - L] if hist % L else [])

    def in_desc(si, b):
        return pltpu.make_async_copy(
            data_hbm.at[wbase + si], data_in.at[b], sem_in[b])

    def ts_desc(si, b):
        return pltpu.make_async_copy(
            ts_hbm.at[pl.ds((wbase + si) * hist, hist)],
            ts_s.at[pl.ds(b * hist, hist)], sem_ts[b])

    def fire_in(si, b):
        in_desc(si, b).start()
        ts_desc(si, b).start()

    def gather_args(si, b):
        return [(pe_hbm.at[idx_s.at[pl.ds(b * hist + off, n)]],
                 pe_rows.at[b, pl.ds(off, n)]) for off, n in gsplit]

    def fire_gathers(si, b):
        ts_desc(si, b).wait()
        for c in cols:
            t = ts_s[pl.ds(b * hist + c, L)].astype(jnp.float32)
            idx_s[pl.ds(b * hist + c, L)] = ((t + 0.5) * 0.01).astype(jnp.int32)
        for src, dst in gather_args(si, b):
            pltpu.async_copy(src, dst, sem_g[b])

    def out_desc(si, b):
        return pltpu.make_async_copy(
            data_in.at[b], out_hbm.at[wbase + si], sem_out[b])

    def fire_addout(si, b):
        for src, dst in gather_args(si, b):
            pltpu.make_async_copy(src, dst, sem_g[b]).wait()
        in_desc(si, b).wait()

        def add_body(r, _):
            for c in range(0, d, L):
                s = pl.ds(c, L)
                data_in[b, r, s] = data_in[b, r, s] + pe_rows[b, r, s]
            return 0
        lax.fori_loop(0, hist, add_body, 0)
        out_desc(si, b).start()

    fire_in(0, 0)
    fire_in(1, 1)
    fire_gathers(0, 0)

    def pair_body(p, _):
        s0 = p * 2
        fire_gathers(s0 + 1, 1)
        fire_addout(s0, 0)
        out_desc(s0, 0).wait()
        fire_in(s0 + 2, 0)
        fire_gathers(s0 + 2, 0)
        fire_addout(s0 + 1, 1)
        out_desc(s0 + 1, 1).wait()
        fire_in(s0 + 3, 1)
        return 0

    lax.fori_loop(0, b_per_w // 2 - 1, pair_body, 0)

    sl = b_per_w - 2
    fire_gathers(sl + 1, 1)
    fire_addout(sl, 0)
    out_desc(sl, 0).wait()
    fire_addout(sl + 1, 1)
    out_desc(sl + 1, 1).wait()


@jax.jit
def kernel(data, time_stamps, pe):
    b, h, d = data.shape

    mesh = plsc.VectorSubcoreMesh(
        core_axis_name="c", subcore_axis_name="s", num_cores=NC,
        num_subcores=NS)
    pe_pad = jnp.pad(pe, ((0, 0), (0, 128 - pe.shape[1])))
    ts_flat = time_stamps.reshape(b * h)
    out = pl.kernel(
        _tec_body,
        out_type=jax.ShapeDtypeStruct((b, h, d), jnp.float32),
        mesh=mesh,
        scratch_types=[
            pltpu.VMEM((2 * h,), jnp.int32),
            pltpu.VMEM((2 * h,), jnp.int32),
            pltpu.VMEM((2, h, d), jnp.float32),
            pltpu.VMEM((2, h, 128), jnp.float32),
            pltpu.SemaphoreType.DMA,
            pltpu.SemaphoreType.DMA,
            pltpu.SemaphoreType.DMA,
            pltpu.SemaphoreType.DMA,
            pltpu.SemaphoreType.DMA,
            pltpu.SemaphoreType.DMA,
            pltpu.SemaphoreType.DMA,
            pltpu.SemaphoreType.DMA,
        ],
    )(data, ts_flat, pe_pad)
    return out


# speedup vs baseline: 3.4106x; 1.0324x over previous
import functools

import jax
import jax.numpy as jnp
from jax import lax
from jax.experimental import pallas as pl
from jax.experimental.pallas import tpu as pltpu
from jax.experimental.pallas import tpu_sc as plsc

NC = 2
NS = 16
NW = NC * NS
L = 16
H0 = 104


def _tec_body(data_hbm, ts_hbm, pe_hbm, out_hbm, ts_s, idx_s,
              data_in0, data_in1, pe_rows0, pe_rows1, pe_sh,
              sem_in0, sem_in1, sem_ts0, sem_ts1, sem_g0, sem_g1,
              sem_out0, sem_out1, sem_pe):
    batch, hist, d = data_hbm.shape
    b_per_w = batch // NW
    wid = lax.axis_index("s") * NC + lax.axis_index("c")
    wbase = wid * b_per_w
    sem_in = (sem_in0, sem_in1)
    sem_ts = (sem_ts0, sem_ts1)
    sem_g = (sem_g0, sem_g1)
    sem_out = (sem_out0, sem_out1)
    data_in = (data_in0, data_in1)
    pe_rows = (pe_rows0, pe_rows1)
    geo = ((0, H0), (H0, hist - H0))

    def in_desc(si, b):
        boff, bn = geo[b]
        return pltpu.make_async_copy(
            data_hbm.at[wbase + si, pl.ds(boff, bn)], data_in[b], sem_in[b])

    def ts_desc(si, b):
        boff, bn = geo[b]
        return pltpu.make_async_copy(
            ts_hbm.at[pl.ds((wbase + si) * hist + boff, bn)],
            ts_s.at[pl.ds(boff, bn)], sem_ts[b])

    def fire_in(si, b):
        in_desc(si, b).start()
        ts_desc(si, b).start()

    def gather_desc(si, b):
        boff, bn = geo[b]
        return pltpu.make_async_copy(
            pe_sh.at[idx_s.at[pl.ds(boff, bn)]], pe_rows[b], sem_g[b])

    def fire_gathers(si, b):
        boff, bn = geo[b]
        ts_desc(si, b).wait()
        cols = list(range(0, bn - L + 1, L))
        if bn % L:
            cols.append(bn - L)
        for c in cols:
            t = ts_s[pl.ds(boff + c, L)].astype(jnp.float32)
            idx_s[pl.ds(boff + c, L)] = ((t + 0.5) * 0.01).astype(jnp.int32)
        gather_desc(si, b).start()

    def out_desc(si, b):
        boff, bn = geo[b]
        return pltpu.make_async_copy(
            data_in[b], out_hbm.at[wbase + si, pl.ds(boff, bn)], sem_out[b])

    def fire_addout(si, b):
        boff, bn = geo[b]
        gather_desc(si, b).wait()
        in_desc(si, b).wait()

        def add_body(r, _):
            for c in range(0, d, L):
                s = pl.ds(c, L)
                data_in[b][r, s] = data_in[b][r, s] + pe_rows[b][r, s]
            return 0
        lax.fori_loop(0, bn, add_body, 0)
        out_desc(si, b).start()

    fire_in(0, 0)
    fire_in(0, 1)
    @pl.when(lax.axis_index("s") == 0)
    def _():
        pltpu.async_copy(pe_hbm, pe_sh, sem_pe).wait()
    plsc.subcore_barrier()
    fire_gathers(0, 0)

    def pair_body(p, _):
        fire_gathers(p, 1)
        fire_addout(p, 0)
        out_desc(p, 0).wait()
        fire_in(p + 1, 0)
        fire_gathers(p + 1, 0)
        fire_addout(p, 1)
        out_desc(p, 1).wait()
        fire_in(p + 1, 1)
        return 0

    lax.fori_loop(0, b_per_w - 1, pair_body, 0)

    pl_ = b_per_w - 1
    fire_gathers(pl_, 1)
    fire_addout(pl_, 0)
    out_desc(pl_, 0).wait()
    fire_addout(pl_, 1)
    out_desc(pl_, 1).wait()


@jax.jit
def kernel(data, time_stamps, pe):
    b, h, d = data.shape

    mesh = plsc.VectorSubcoreMesh(
        core_axis_name="c", subcore_axis_name="s", num_cores=NC,
        num_subcores=NS)
    pe_pad = jnp.pad(pe, ((0, 0), (0, 128 - pe.shape[1])))
    ts_flat = time_stamps.reshape(b * h)
    out = pl.kernel(
        _tec_body,
        out_type=jax.ShapeDtypeStruct((b, h, d), jnp.float32),
        mesh=mesh,
        scratch_types=[
            pltpu.VMEM((h,), jnp.int32),
            pltpu.VMEM((h,), jnp.int32),
            pltpu.VMEM((H0, d), jnp.float32),
            pltpu.VMEM((h - H0, d), jnp.float32),
            pltpu.VMEM((H0, 128), jnp.float32),
            pltpu.VMEM((h - H0, 128), jnp.float32),
            pltpu.VMEM_SHARED(pe_pad.shape, jnp.float32),
            pltpu.SemaphoreType.DMA,
            pltpu.SemaphoreType.DMA,
            pltpu.SemaphoreType.DMA,
            pltpu.SemaphoreType.DMA,
            pltpu.SemaphoreType.DMA,
            pltpu.SemaphoreType.DMA,
            pltpu.SemaphoreType.DMA,
            pltpu.SemaphoreType.DMA,
            pltpu.SemaphoreType.DMA,
        ],
    )(data, ts_flat, pe_pad)
    return out


# speedup vs baseline: 4.1783x; 1.2251x over previous
import functools

import jax
import jax.numpy as jnp
from jax import lax
from jax.experimental import pallas as pl
from jax.experimental.pallas import tpu as pltpu
from jax.experimental.pallas import tpu_sc as plsc

NC = 2
NS = 16
NW = NC * NS
L = 16
DBLK = 8


def _tec_body(data_hbm, ts_hbm, pe_hbm, out_hbm, pe_c, ts_v, idx_v,
              data_v0, data_v1, sem_in0, sem_in1, sem_ts0, sem_ts1,
              sem_out0, sem_out1, sem_pe):
    hist, d, batch = data_hbm.shape
    v = pe_hbm.shape[0] // d
    ngrp = d // DBLK
    nq = NW // ngrp
    bq = batch // nq
    wid = lax.axis_index("s") * NC + lax.axis_index("c")
    g = wid // nq
    q = wid % nq
    dbase = g * DBLK
    bbase = q * bq
    sem_in = (sem_in0, sem_in1)
    sem_ts = (sem_ts0, sem_ts1)
    sem_out = (sem_out0, sem_out1)
    data_v = (data_v0, data_v1)

    def in_desc(h, b):
        return pltpu.make_async_copy(
            data_hbm.at[h, pl.ds(dbase, DBLK), pl.ds(bbase, bq)],
            data_v[b], sem_in[b])

    def ts_desc(h, b):
        return pltpu.make_async_copy(
            ts_hbm.at[pl.ds(h * batch + bbase, bq)],
            ts_v.at[pl.ds(b * bq, bq)], sem_ts[b])

    def fire_in(h, b):
        in_desc(h, b).start()
        ts_desc(h, b).start()

    def out_desc(h, b):
        return pltpu.make_async_copy(
            data_v[b],
            out_hbm.at[h, pl.ds(dbase, DBLK), pl.ds(bbase, bq)], sem_out[b])

    def fire_work(h, b):
        ts_desc(h, b).wait()
        def div_body(i, _):
            s = pl.ds(b * bq + i * L, L)
            t = ts_v[s].astype(jnp.float32)
            idx_v[s] = ((t + 0.5) * 0.01).astype(jnp.int32)
            return 0
        lax.fori_loop(0, bq // L, div_body, 0)
        in_desc(h, b).wait()

        def add_body(i, _):
            iv = idx_v[pl.ds(b * bq + i * L, L)]
            for dl in range(DBLK):
                rows = plsc.load_gather(pe_c, [iv + (dl * v)])
                s = pl.ds(i * L, L)
                data_v[b][dl, s] = data_v[b][dl, s] + rows
            return 0
        lax.fori_loop(0, bq // L, add_body, 0)
        out_desc(h, b).start()

    for dl in range(DBLK):
        pltpu.async_copy(
            pe_hbm.at[pl.ds((dbase + dl) * v, v)],
            pe_c.at[pl.ds(dl * v, v)], sem_pe).wait()

    fire_in(0, 0)
    fire_in(1, 1)
    fire_work(0, 0)

    def pair_body(p, _):
        h0 = p * 2
        out_desc(h0, 0).wait()
        fire_in(h0 + 2, 0)
        fire_work(h0 + 1, 1)
        out_desc(h0 + 1, 1).wait()
        fire_in(h0 + 3, 1)
        fire_work(h0 + 2, 0)
        return 0

    lax.fori_loop(0, hist // 2 - 1, pair_body, 0)

    hl = hist - 2
    out_desc(hl, 0).wait()
    fire_work(hl + 1, 1)
    out_desc(hl + 1, 1).wait()


@jax.jit
def kernel(data, time_stamps, pe):
    b, h, d = data.shape
    v = pe.shape[0]
    data_t = jnp.transpose(data, (1, 2, 0))
    ts_flat = jnp.transpose(time_stamps, (1, 0)).reshape(h * b)
    pe_flat = jnp.transpose(pe, (1, 0)).reshape(d * v)

    mesh = plsc.VectorSubcoreMesh(
        core_axis_name="c", subcore_axis_name="s", num_cores=NC,
        num_subcores=NS)
    bq = b // (NW // (d // DBLK))
    out_t = pl.kernel(
        _tec_body,
        out_type=jax.ShapeDtypeStruct((h, d, b), jnp.float32),
        mesh=mesh,
        scratch_types=[
            pltpu.VMEM((DBLK * v,), jnp.float32),
            pltpu.VMEM((2 * bq,), jnp.int32),
            pltpu.VMEM((2 * bq,), jnp.int32),
            pltpu.VMEM((DBLK, bq), jnp.float32),
            pltpu.VMEM((DBLK, bq), jnp.float32),
            pltpu.SemaphoreType.DMA,
            pltpu.SemaphoreType.DMA,
            pltpu.SemaphoreType.DMA,
            pltpu.SemaphoreType.DMA,
            pltpu.SemaphoreType.DMA,
            pltpu.SemaphoreType.DMA,
            pltpu.SemaphoreType.DMA,
        ],
        compiler_params=pltpu.CompilerParams(needs_layout_passes=False),
    )(data_t, ts_flat, pe_flat)
    return jnp.transpose(out_t, (2, 0, 1))


# speedup vs baseline: 4.2529x; 1.0178x over previous
import functools

import jax
import jax.numpy as jnp
from jax import lax
from jax.experimental import pallas as pl
from jax.experimental.pallas import tpu as pltpu
from jax.experimental.pallas import tpu_sc as plsc

NC = 2
NS = 16
NW = NC * NS
L = 16
DBLK = 8


def _tec_body(data_hbm, ts_hbm, pe_hbm, out_hbm, pe_c, ts_v,
              data_v0, data_v1, sem_in0, sem_in1, sem_ts0, sem_ts1,
              sem_out0, sem_out1, sem_pe):
    hist, d, batch = data_hbm.shape
    v = pe_hbm.shape[0] // d
    ngrp = d // DBLK
    nq = NW // ngrp
    bq = batch // nq
    wid = lax.axis_index("s") * NC + lax.axis_index("c")
    g = wid // nq
    q = wid % nq
    dbase = g * DBLK
    bbase = q * bq
    sem_in = (sem_in0, sem_in1)
    sem_ts = (sem_ts0, sem_ts1)
    sem_out = (sem_out0, sem_out1)
    data_v = (data_v0, data_v1)

    def in_desc(h, b):
        return pltpu.make_async_copy(
            data_hbm.at[h, pl.ds(dbase, DBLK), pl.ds(bbase, bq)],
            data_v[b], sem_in[b])

    def ts_desc(h, b):
        return pltpu.make_async_copy(
            ts_hbm.at[pl.ds(h * batch + bbase, bq)],
            ts_v.at[pl.ds(b * bq, bq)], sem_ts[b])

    def fire_in(h, b):
        in_desc(h, b).start()
        ts_desc(h, b).start()

    def out_desc(h, b):
        return pltpu.make_async_copy(
            data_v[b],
            out_hbm.at[h, pl.ds(dbase, DBLK), pl.ds(bbase, bq)], sem_out[b])

    def fire_work(h, b):
        ts_desc(h, b).wait()
        in_desc(h, b).wait()

        def add_body(i, _):
            for u in range(2):
                s16 = pl.ds((i * 2 + u) * L, L)
                t = ts_v[pl.ds(b * bq + (i * 2 + u) * L, L)]
                iv = ((t.astype(jnp.float32) + 0.5) * 0.01).astype(jnp.int32)
                for dl in range(DBLK):
                    rows = plsc.load_gather(pe_c, [iv + (dl * v)])
                    data_v[b][dl, s16] = data_v[b][dl, s16] + rows
            return 0
        lax.fori_loop(0, bq // L // 2, add_body, 0)
        out_desc(h, b).start()

    for dl in range(DBLK):
        pltpu.async_copy(
            pe_hbm.at[pl.ds((dbase + dl) * v, v)],
            pe_c.at[pl.ds(dl * v, v)], sem_pe).wait()

    fire_in(0, 0)
    fire_in(1, 1)
    fire_work(0, 0)

    def pair_body(p, _):
        h0 = p * 2
        out_desc(h0, 0).wait()
        fire_in(h0 + 2, 0)
        fire_work(h0 + 1, 1)
        out_desc(h0 + 1, 1).wait()
        fire_in(h0 + 3, 1)
        fire_work(h0 + 2, 0)
        return 0

    lax.fori_loop(0, hist // 2 - 1, pair_body, 0)

    hl = hist - 2
    out_desc(hl, 0).wait()
    fire_work(hl + 1, 1)
    out_desc(hl + 1, 1).wait()


@jax.jit
def kernel(data, time_stamps, pe):
    b, h, d = data.shape
    v = pe.shape[0]
    data_t = jnp.transpose(data, (1, 2, 0))
    ts_flat = jnp.transpose(time_stamps, (1, 0)).reshape(h * b)
    pe_flat = jnp.transpose(pe, (1, 0)).reshape(d * v)

    mesh = plsc.VectorSubcoreMesh(
        core_axis_name="c", subcore_axis_name="s", num_cores=NC,
        num_subcores=NS)
    bq = b // (NW // (d // DBLK))
    out_t = pl.kernel(
        _tec_body,
        out_type=jax.ShapeDtypeStruct((h, d, b), jnp.float32),
        mesh=mesh,
        scratch_types=[
            pltpu.VMEM((DBLK * v,), jnp.float32),
            pltpu.VMEM((2 * bq,), jnp.int32),
            pltpu.VMEM((DBLK, bq), jnp.float32),
            pltpu.VMEM((DBLK, bq), jnp.float32),
            pltpu.SemaphoreType.DMA,
            pltpu.SemaphoreType.DMA,
            pltpu.SemaphoreType.DMA,
            pltpu.SemaphoreType.DMA,
            pltpu.SemaphoreType.DMA,
            pltpu.SemaphoreType.DMA,
            pltpu.SemaphoreType.DMA,
        ],
        compiler_params=pltpu.CompilerParams(needs_layout_passes=False),
    )(data_t, ts_flat, pe_flat)
    return jnp.transpose(out_t, (2, 0, 1))


# speedup vs baseline: 12.2237x; 2.8742x over previous
import functools

import jax
import jax.numpy as jnp
from jax import lax
from jax.experimental import pallas as pl
from jax.experimental.pallas import tpu as pltpu
from jax.experimental.pallas import tpu_sc as plsc

NC = 2
NS = 16
NW = NC * NS
L = 16
DBLK = 8


def _tec_body(data_hbm, ts_hbm, pe_hbm, out_hbm, pe_c, ts_v,
              data_v0, data_v1, sem_in0, sem_in1, sem_ts0, sem_ts1,
              sem_out0, sem_out1, sem_pe):
    hist, d, batch = data_hbm.shape
    v = pe_hbm.shape[0] // d
    ngrp = d // DBLK
    nq = NW // ngrp
    bq = batch // nq
    wid = lax.axis_index("s") * NC + lax.axis_index("c")
    g = wid // nq
    q = wid % nq
    dbase = g * DBLK
    bbase = q * bq
    sem_in = (sem_in0, sem_in1)
    sem_ts = (sem_ts0, sem_ts1)
    sem_out = (sem_out0, sem_out1)
    data_v = (data_v0, data_v1)

    def in_desc(h, b):
        return pltpu.make_async_copy(
            data_hbm.at[h, pl.ds(dbase, DBLK), pl.ds(bbase, bq)],
            data_v[b], sem_in[b])

    def ts_desc(h, b):
        return pltpu.make_async_copy(
            ts_hbm.at[pl.ds(h * batch + bbase, bq)],
            ts_v.at[pl.ds(b * bq, bq)], sem_ts[b])

    def fire_in(h, b):
        in_desc(h, b).start()
        ts_desc(h, b).start()

    def out_desc(h, b):
        return pltpu.make_async_copy(
            data_v[b],
            out_hbm.at[h, pl.ds(dbase, DBLK), pl.ds(bbase, bq)], sem_out[b])

    def fire_work(h, b):
        ts_desc(h, b).wait()
        in_desc(h, b).wait()

        @plsc.parallel_loop(0, bq // L, 1, unroll=2)
        def add_body(i):
            s16 = pl.ds(i * L, L)
            t = ts_v[pl.ds(b * bq + i * L, L)]
            iv = ((t.astype(jnp.float32) + 0.5) * 0.01).astype(jnp.int32)
            for dl in range(DBLK):
                rows = plsc.load_gather(pe_c, [iv + (dl * v)])
                data_v[b][dl, s16] = data_v[b][dl, s16] + rows
        out_desc(h, b).start()

    for dl in range(DBLK):
        pltpu.async_copy(
            pe_hbm.at[pl.ds((dbase + dl) * v, v)],
            pe_c.at[pl.ds(dl * v, v)], sem_pe).wait()

    fire_in(0, 0)
    fire_in(1, 1)
    fire_work(0, 0)

    def pair_body(p, _):
        h0 = p * 2
        out_desc(h0, 0).wait()
        fire_in(h0 + 2, 0)
        fire_work(h0 + 1, 1)
        out_desc(h0 + 1, 1).wait()
        fire_in(h0 + 3, 1)
        fire_work(h0 + 2, 0)
        return 0

    lax.fori_loop(0, hist // 2 - 1, pair_body, 0)

    hl = hist - 2
    out_desc(hl, 0).wait()
    fire_work(hl + 1, 1)
    out_desc(hl + 1, 1).wait()


@jax.jit
def kernel(data, time_stamps, pe):
    b, h, d = data.shape
    v = pe.shape[0]
    data_t = jnp.transpose(data, (1, 2, 0))
    ts_flat = jnp.transpose(time_stamps, (1, 0)).reshape(h * b)
    pe_flat = jnp.transpose(pe, (1, 0)).reshape(d * v)

    mesh = plsc.VectorSubcoreMesh(
        core_axis_name="c", subcore_axis_name="s", num_cores=NC,
        num_subcores=NS)
    bq = b // (NW // (d // DBLK))
    out_t = pl.kernel(
        _tec_body,
        out_type=jax.ShapeDtypeStruct((h, d, b), jnp.float32),
        mesh=mesh,
        scratch_types=[
            pltpu.VMEM((DBLK * v,), jnp.float32),
            pltpu.VMEM((2 * bq,), jnp.int32),
            pltpu.VMEM((DBLK, bq), jnp.float32),
            pltpu.VMEM((DBLK, bq), jnp.float32),
            pltpu.SemaphoreType.DMA,
            pltpu.SemaphoreType.DMA,
            pltpu.SemaphoreType.DMA,
            pltpu.SemaphoreType.DMA,
            pltpu.SemaphoreType.DMA,
            pltpu.SemaphoreType.DMA,
            pltpu.SemaphoreType.DMA,
        ],
        compiler_params=pltpu.CompilerParams(needs_layout_passes=False),
    )(data_t, ts_flat, pe_flat)
    return jnp.transpose(out_t, (2, 0, 1))


# speedup vs baseline: 12.3119x; 1.0072x over previous
import functools

import jax
import jax.numpy as jnp
from jax import lax
from jax.experimental import pallas as pl
from jax.experimental.pallas import tpu as pltpu
from jax.experimental.pallas import tpu_sc as plsc

NC = 2
NS = 16
NW = NC * NS
L = 16
DBLK = 8


def _tec_body(data_hbm, ts_hbm, pe_hbm, out_hbm, pe_c, ts_v,
              data_v0, data_v1, sem_in0, sem_in1, sem_ts0, sem_ts1,
              sem_out0, sem_out1, sem_pe):
    hist, d, batch = data_hbm.shape
    v = pe_hbm.shape[0] // d
    ngrp = d // DBLK
    nq = NW // ngrp
    bq = batch // nq
    wid = lax.axis_index("s") * NC + lax.axis_index("c")
    g = wid // nq
    q = wid % nq
    dbase = g * DBLK
    bbase = q * bq
    sem_in = (sem_in0, sem_in1)
    sem_ts = (sem_ts0, sem_ts1)
    sem_out = (sem_out0, sem_out1)
    data_v = (data_v0, data_v1)

    def in_desc(h, b):
        return pltpu.make_async_copy(
            data_hbm.at[h, pl.ds(dbase, DBLK), pl.ds(bbase, bq)],
            data_v[b], sem_in[b])

    def ts_desc(h, b):
        return pltpu.make_async_copy(
            ts_hbm.at[pl.ds(h * batch + bbase, bq)],
            ts_v.at[pl.ds(b * bq, bq)], sem_ts[b])

    def fire_in(h, b):
        in_desc(h, b).start()
        ts_desc(h, b).start()

    def out_desc(h, b):
        return pltpu.make_async_copy(
            data_v[b],
            out_hbm.at[h, pl.ds(dbase, DBLK), pl.ds(bbase, bq)], sem_out[b])

    def fire_work(h, b):
        ts_desc(h, b).wait()
        in_desc(h, b).wait()

        @plsc.parallel_loop(0, bq // L, 1, unroll=4)
        def add_body(i):
            s16 = pl.ds(i * L, L)
            t = ts_v[pl.ds(b * bq + i * L, L)]
            iv = ((t.astype(jnp.float32) + 0.5) * 0.01).astype(jnp.int32)
            for dl in range(DBLK):
                rows = plsc.load_gather(pe_c, [iv + (dl * v)])
                data_v[b][dl, s16] = data_v[b][dl, s16] + rows
        out_desc(h, b).start()

    for dl in range(DBLK):
        pltpu.async_copy(
            pe_hbm.at[pl.ds((dbase + dl) * v, v)],
            pe_c.at[pl.ds(dl * v, v)], sem_pe).wait()

    fire_in(0, 0)
    fire_in(1, 1)
    fire_work(0, 0)

    def pair_body(p, _):
        h0 = p * 2
        out_desc(h0, 0).wait()
        fire_in(h0 + 2, 0)
        fire_work(h0 + 1, 1)
        out_desc(h0 + 1, 1).wait()
        fire_in(h0 + 3, 1)
        fire_work(h0 + 2, 0)
        return 0

    lax.fori_loop(0, hist // 2 - 1, pair_body, 0)

    hl = hist - 2
    out_desc(hl, 0).wait()
    fire_work(hl + 1, 1)
    out_desc(hl + 1, 1).wait()


@jax.jit
def kernel(data, time_stamps, pe):
    b, h, d = data.shape
    v = pe.shape[0]
    data_t = jnp.transpose(data, (1, 2, 0))
    ts_flat = jnp.transpose(time_stamps, (1, 0)).reshape(h * b)
    pe_flat = jnp.transpose(pe, (1, 0)).reshape(d * v)

    mesh = plsc.VectorSubcoreMesh(
        core_axis_name="c", subcore_axis_name="s", num_cores=NC,
        num_subcores=NS)
    bq = b // (NW // (d // DBLK))
    out_t = pl.kernel(
        _tec_body,
        out_type=jax.ShapeDtypeStruct((h, d, b), jnp.float32),
        mesh=mesh,
        scratch_types=[
            pltpu.VMEM((DBLK * v,), jnp.float32),
            pltpu.VMEM((2 * bq,), jnp.int32),
            pltpu.VMEM((DBLK, bq), jnp.float32),
            pltpu.VMEM((DBLK, bq), jnp.float32),
            pltpu.SemaphoreType.DMA,
            pltpu.SemaphoreType.DMA,
            pltpu.SemaphoreType.DMA,
            pltpu.SemaphoreType.DMA,
            pltpu.SemaphoreType.DMA,
            pltpu.SemaphoreType.DMA,
            pltpu.SemaphoreType.DMA,
        ],
        compiler_params=pltpu.CompilerParams(needs_layout_passes=False),
    )(data_t, ts_flat, pe_flat)
    return jnp.transpose(out_t, (2, 0, 1))
